# baseline (speedup 1.0000x reference)
import functools

import jax
import jax.numpy as jnp
from jax import lax
from jax.experimental import pallas as pl
from jax.experimental.pallas import tpu as pltpu
from jax.experimental.pallas import tpu_sc as plsc

_NB = 16


@functools.partial(jax.jit, static_argnames=("nb",))
def _gather_sc(v, idx, nb):
    b, h = idx.shape
    d = v.shape[1]
    info = plsc.get_sparse_core_info()
    nw = info.num_cores * info.num_subcores
    rows_per_w = b // nw
    n_chunks = rows_per_w // nb

    mesh = plsc.VectorSubcoreMesh(core_axis_name="c", subcore_axis_name="s")

    @functools.partial(
        pl.kernel,
        mesh=mesh,
        out_type=jax.ShapeDtypeStruct((b, h, d), jnp.float32),
        compiler_params=pltpu.CompilerParams(use_tc_tiling_on_sc=False),
        scratch_types=[
            pltpu.VMEM((rows_per_w, h), jnp.int32),
            pltpu.VMEM((2, nb, h, d), jnp.float32),
            pltpu.SemaphoreType.DMA,
            pltpu.SemaphoreType.DMA,
            pltpu.SemaphoreType.DMA,
            pltpu.SemaphoreType.DMA,
        ],
    )
    def k(table_hbm, idx_hbm, out_hbm, idx_all, rows_v, gsem0, gsem1,
          wsem0, wsem1):
        wid = lax.axis_index("s") * info.num_cores + lax.axis_index("c")
        base = wid * rows_per_w
        pltpu.sync_copy(idx_hbm.at[pl.ds(base, rows_per_w)], idx_all)

        gsem = (gsem0, gsem1)
        wsem = (wsem0, wsem1)

        def start_gathers(g, slot):
            for j in range(nb):
                pltpu.make_async_copy(
                    table_hbm.at[idx_all.at[g * nb + j]],
                    rows_v.at[slot, j],
                    gsem[slot],
                ).start()

        def wait_gathers(slot):
            pltpu.make_async_copy(
                table_hbm.at[idx_all.at[0]],
                rows_v.at[slot],
                gsem[slot],
            ).wait()

        def start_write(g, slot):
            pltpu.make_async_copy(
                rows_v.at[slot],
                out_hbm.at[pl.ds(base + g * nb, nb)],
                wsem[slot],
            ).start()

        def wait_write(slot):
            pltpu.make_async_copy(
                rows_v.at[slot],
                out_hbm.at[pl.ds(base, nb)],
                wsem[slot],
            ).wait()

        start_gathers(0, 0)

        start_gathers(1, 1)
        wait_gathers(0)
        start_write(0, 0)
        wait_write(0)
        start_gathers(2, 0)
        wait_gathers(1)
        start_write(1, 1)

        def body(i, carry):
            a = 2 * i
            wait_write(1)
            start_gathers(a + 1, 1)
            wait_gathers(0)
            start_write(a, 0)
            wait_write(0)
            start_gathers(a + 2, 0)
            wait_gathers(1)
            start_write(a + 1, 1)
            return carry

        lax.fori_loop(1, n_chunks // 2 - 1, body, 0)

        a = n_chunks - 2
        wait_write(1)
        start_gathers(a + 1, 1)
        wait_gathers(0)
        start_write(a, 0)
        wait_write(0)
        wait_gathers(1)
        start_write(a + 1, 1)
        wait_write(1)

    return k(v, idx)


def kernel(v, idx):
    return _gather_sc(v, idx, _NB)
